# baseline (speedup 1.0000x reference)
import functools

import jax
import jax.numpy as jnp
from jax import lax
from jax.experimental import pallas as pl
from jax.experimental.pallas import tpu as pltpu
from jax.experimental.pallas import tpu_sc as plsc

N = 10000
E = 160000
D = 256
HALF = 128

NC = 2
NS = 16
EPT = E // NS
CHUNK = 32
EPTP = 10240
NCHUNK = EPTP // CHUNK
NP = 10240
RPT = NP // NS


def _matmul_body(x_ref, w_ref, b_ref, out_ref):
    y = jnp.dot(x_ref[...], w_ref[...], preferred_element_type=jnp.float32)
    y = y + b_ref[0][None, :]
    out_ref[0] = y[:, :HALF]
    out_ref[1] = y[:, HALF:]


def _matmul(x, W, b):
    BR = 1000
    grid = (N // BR,)
    return pl.pallas_call(
        _matmul_body,
        grid=grid,
        in_specs=[
            pl.BlockSpec((BR, D), lambda i: (i, 0)),
            pl.BlockSpec((D, D), lambda i: (0, 0)),
            pl.BlockSpec((1, D), lambda i: (0, 0)),
        ],
        out_specs=pl.BlockSpec((2, BR, HALF), lambda i: (0, i, 0)),
        out_shape=jax.ShapeDtypeStruct((2, NP, HALF), jnp.float32),
    )(x, W, b.reshape(1, D))


NBUF = 8
NHALF = 8
NCHUNKH = NCHUNK // NHALF
NGROUPH = NCHUNKH // NBUF


def _sc_scatter_body(xt_hbm, src_hbm, dst_hbm, out_hbm,
                     src_v, dst_v, bufs, acc, gsem, ssem):
    c = lax.axis_index("c")
    s = lax.axis_index("s")
    seed_base = c * NP + s * RPT
    pltpu.sync_copy(xt_hbm.at[pl.ds(seed_base, RPT)],
                    acc.at[pl.ds(s * RPT, RPT)])
    plsc.subcore_barrier()

    def gather(j, b):
        return pltpu.async_copy(xt_hbm.at[src_v.at[j]], bufs.at[b],
                                gsem.at[b])

    for h in range(NHALF):
        pltpu.sync_copy(src_hbm.at[c, s, h], src_v)
        pltpu.sync_copy(dst_hbm.at[s, h], dst_v)

        for b in range(NBUF):
            gather(b, b)

        def group(g, carry):
            j0 = g * NBUF
            for b in range(NBUF):
                pltpu.make_async_copy(xt_hbm.at[src_v.at[j0 + b]],
                                      bufs.at[b], gsem.at[b]).wait()
                pltpu.async_copy(bufs.at[b], acc.at[dst_v.at[j0 + b]],
                                 ssem.at[b], add=True)
            for b in range(NBUF):
                pltpu.make_async_copy(bufs.at[b], acc.at[dst_v.at[j0 + b]],
                                      ssem.at[b]).wait()

                @pl.when(g + 1 < NGROUPH)
                def _():
                    gather(j0 + NBUF + b, b)

            return carry

        lax.fori_loop(0, NGROUPH, group, 0)

    plsc.subcore_barrier()
    pltpu.sync_copy(acc.at[pl.ds(s * RPT, RPT)],
                    out_hbm.at[pl.ds(seed_base, RPT)])


@functools.partial(jax.jit, static_argnames=())
def _sc_scatter(xt_flat, src3, dst3):
    mesh = plsc.VectorSubcoreMesh(core_axis_name="c", subcore_axis_name="s")
    run = pl.kernel(
        _sc_scatter_body,
        out_type=jax.ShapeDtypeStruct((NC * NP, HALF), jnp.float32),
        mesh=mesh,
        scratch_types=[
            pltpu.VMEM((NCHUNKH, CHUNK), jnp.int32),
            pltpu.VMEM((NCHUNKH, CHUNK), jnp.int32),
            pltpu.VMEM((NBUF, CHUNK, HALF), jnp.float32),
            pltpu.VMEM_SHARED((NP, HALF), jnp.float32),
            pltpu.SemaphoreType.DMA((NBUF,)),
            pltpu.SemaphoreType.DMA((NBUF,)),
        ],
    )
    return run(xt_flat, src3, dst3)


def _ln_body(agg_ref, g_ref, bt_ref, out_ref):
    xa = jnp.concatenate([agg_ref[0], agg_ref[1]], axis=1)
    mean = jnp.mean(xa, axis=1, keepdims=True)
    d = xa - mean
    var = jnp.mean(d * d, axis=1, keepdims=True)
    y = d * lax.rsqrt(var + 1e-5) * g_ref[0][None, :] + bt_ref[0][None, :]
    out_ref[...] = jnp.maximum(y, 0.0)


def _layernorm(agg2, gamma, beta):
    BR = 1000
    grid = (N // BR,)
    return pl.pallas_call(
        _ln_body,
        grid=grid,
        in_specs=[
            pl.BlockSpec((2, BR, HALF), lambda i: (0, i, 0)),
            pl.BlockSpec((1, D), lambda i: (0, 0)),
            pl.BlockSpec((1, D), lambda i: (0, 0)),
        ],
        out_specs=pl.BlockSpec((BR, D), lambda i: (i, 0)),
        out_shape=jax.ShapeDtypeStruct((N, D), jnp.float32),
    )(agg2, gamma.reshape(1, D), beta.reshape(1, D))


def kernel(x, edge_index, W, b, gamma, beta):
    src = edge_index[0].astype(jnp.int32)
    dst = edge_index[1].astype(jnp.int32)
    xt2 = _matmul(x, W, b)
    pad = EPTP - EPT
    srcp = jnp.concatenate(
        [src.reshape(NS, EPT), jnp.zeros((NS, pad), jnp.int32)], axis=1)
    pad_rows = N + jnp.arange(pad, dtype=jnp.int32) % (NP - N)
    dstp = jnp.concatenate(
        [dst.reshape(NS, EPT),
         jnp.broadcast_to(pad_rows, (NS, pad))], axis=1)
    src3 = jnp.stack([srcp, srcp + NP]).reshape(NC, NS, NHALF, NCHUNKH, CHUNK)
    dst3 = dstp.reshape(NS, NHALF, NCHUNKH, CHUNK)
    agg = _sc_scatter(xt2.reshape(NC * NP, HALF), src3, dst3)
    return _layernorm(agg.reshape(NC, NP, HALF), gamma, beta)


# speedup vs baseline: 1.6905x; 1.6905x over previous
import functools

import jax
import jax.numpy as jnp
from jax import lax
from jax.experimental import pallas as pl
from jax.experimental.pallas import tpu as pltpu
from jax.experimental.pallas import tpu_sc as plsc

N = 10000
E = 160000
D = 256
HALF = 128

NC = 2
NS = 16
EPT = E // NS
CHUNK = 32
EPTP = 10240
NCHUNK = EPTP // CHUNK
NP = 10240
RPT = NP // NS


def _matmul_body(x_ref, w_ref, b_ref, out_ref):
    y = jnp.dot(x_ref[...], w_ref[...], preferred_element_type=jnp.float32)
    y = y + b_ref[0][None, :]
    out_ref[0] = y[:, :HALF]
    out_ref[1] = y[:, HALF:]


def _matmul(x, W, b):
    BR = 1000
    grid = (N // BR,)
    return pl.pallas_call(
        _matmul_body,
        grid=grid,
        in_specs=[
            pl.BlockSpec((BR, D), lambda i: (i, 0)),
            pl.BlockSpec((D, D), lambda i: (0, 0)),
            pl.BlockSpec((1, D), lambda i: (0, 0)),
        ],
        out_specs=pl.BlockSpec((2, BR, HALF), lambda i: (0, i, 0)),
        out_shape=jax.ShapeDtypeStruct((2, NP, HALF), jnp.float32),
    )(x, W, b.reshape(1, D))


NBUF = 8
NHALF = 8
NCHUNKH = NCHUNK // NHALF
NGROUPH = NCHUNKH // NBUF


def _sc_scatter_body(xt_hbm, src_hbm, dst_hbm, out_hbm,
                     src_v, dst_v, bufs, acc, gsem, ssem):
    c = lax.axis_index("c")
    s = lax.axis_index("s")
    seed_base = c * NP + s * RPT
    pltpu.sync_copy(xt_hbm.at[pl.ds(seed_base, RPT)],
                    acc.at[pl.ds(s * RPT, RPT)])
    plsc.subcore_barrier()

    def gather(j, b):
        return pltpu.async_copy(xt_hbm.at[src_v.at[j]], bufs.at[b],
                                gsem.at[b])

    for h in range(NHALF):
        pltpu.sync_copy(src_hbm.at[c, s, h], src_v)
        pltpu.sync_copy(dst_hbm.at[s, h], dst_v)

        for b in range(NBUF):
            gather(b, b)

        def group(g, carry):
            j0 = g * NBUF
            for b in range(NBUF):
                pltpu.make_async_copy(xt_hbm.at[src_v.at[j0 + b]],
                                      bufs.at[b], gsem.at[b]).wait()
                pltpu.async_copy(bufs.at[b], acc.at[dst_v.at[j0 + b]],
                                 ssem.at[b], add=True)
            for b in range(NBUF):
                pltpu.make_async_copy(bufs.at[b], acc.at[dst_v.at[j0 + b]],
                                      ssem.at[b]).wait()

                @pl.when(g + 1 < NGROUPH)
                def _():
                    gather(j0 + NBUF + b, b)

            return carry

        lax.fori_loop(0, NGROUPH, group, 0)

    plsc.subcore_barrier()
    pltpu.sync_copy(acc.at[pl.ds(s * RPT, RPT)],
                    out_hbm.at[pl.ds(seed_base, RPT)])


@functools.partial(jax.jit, static_argnames=())
def _sc_scatter(xt_flat, src3, dst3):
    mesh = plsc.VectorSubcoreMesh(core_axis_name="c", subcore_axis_name="s")
    run = pl.kernel(
        _sc_scatter_body,
        out_type=jax.ShapeDtypeStruct((NC * NP, HALF), jnp.float32),
        mesh=mesh,
        scratch_types=[
            pltpu.VMEM((NCHUNKH, CHUNK), jnp.int32),
            pltpu.VMEM((NCHUNKH, CHUNK), jnp.int32),
            pltpu.VMEM((NBUF, CHUNK, HALF), jnp.float32),
            pltpu.VMEM_SHARED((NP, HALF), jnp.float32),
            pltpu.SemaphoreType.DMA((NBUF,)),
            pltpu.SemaphoreType.DMA((NBUF,)),
        ],
    )
    return run(xt_flat, src3, dst3)


def _ln_body(agg_ref, g_ref, bt_ref, out_ref):
    xa = jnp.concatenate([agg_ref[0], agg_ref[1]], axis=1)
    mean = jnp.mean(xa, axis=1, keepdims=True)
    d = xa - mean
    var = jnp.mean(d * d, axis=1, keepdims=True)
    y = d * lax.rsqrt(var + 1e-5) * g_ref[0][None, :] + bt_ref[0][None, :]
    out_ref[...] = jnp.maximum(y, 0.0)


def _layernorm(agg2, gamma, beta):
    BR = 1000
    grid = (N // BR,)
    return pl.pallas_call(
        _ln_body,
        grid=grid,
        in_specs=[
            pl.BlockSpec((2, BR, HALF), lambda i: (0, i, 0)),
            pl.BlockSpec((1, D), lambda i: (0, 0)),
            pl.BlockSpec((1, D), lambda i: (0, 0)),
        ],
        out_specs=pl.BlockSpec((BR, D), lambda i: (i, 0)),
        out_shape=jax.ShapeDtypeStruct((N, D), jnp.float32),
    )(agg2, gamma.reshape(1, D), beta.reshape(1, D))


def kernel(x, edge_index, W, b, gamma, beta):
    src = edge_index[0].astype(jnp.int32)
    dst = edge_index[1].astype(jnp.int32)
    xt2 = _matmul(x, W, b)
    pad = EPTP - EPT
    pad_src = jnp.arange(pad, dtype=jnp.int32) * 37 % N
    srcp = jnp.concatenate(
        [src.reshape(NS, EPT), jnp.broadcast_to(pad_src, (NS, pad))], axis=1)
    pad_rows = N + jnp.arange(pad, dtype=jnp.int32) % (NP - N)
    dstp = jnp.concatenate(
        [dst.reshape(NS, EPT),
         jnp.broadcast_to(pad_rows, (NS, pad))], axis=1)
    src3 = jnp.stack([srcp, srcp + NP]).reshape(NC, NS, NHALF, NCHUNKH, CHUNK)
    dst3 = dstp.reshape(NS, NHALF, NCHUNKH, CHUNK)
    agg = _sc_scatter(xt2.reshape(NC * NP, HALF), src3, dst3)
    return _layernorm(agg.reshape(NC, NP, HALF), gamma, beta)


# speedup vs baseline: 1.7436x; 1.0314x over previous
import functools

import jax
import jax.numpy as jnp
from jax import lax
from jax.experimental import pallas as pl
from jax.experimental.pallas import tpu as pltpu
from jax.experimental.pallas import tpu_sc as plsc

N = 10000
E = 160000
D = 256
HALF = 128

NC = 2
NS = 16
EPT = E // NS
CHUNK = 40
EPTP = 10080
NCHUNK = EPTP // CHUNK
NP = 10240
RPT = NP // NS


def _matmul_body(x_ref, w_ref, b_ref, out_ref):
    y = jnp.dot(x_ref[...], w_ref[...], preferred_element_type=jnp.float32)
    y = y + b_ref[0][None, :]
    out_ref[0] = y[:, :HALF]
    out_ref[1] = y[:, HALF:]


def _matmul(x, W, b):
    BR = 1000
    grid = (N // BR,)
    return pl.pallas_call(
        _matmul_body,
        grid=grid,
        in_specs=[
            pl.BlockSpec((BR, D), lambda i: (i, 0)),
            pl.BlockSpec((D, D), lambda i: (0, 0)),
            pl.BlockSpec((1, D), lambda i: (0, 0)),
        ],
        out_specs=pl.BlockSpec((2, BR, HALF), lambda i: (0, i, 0)),
        out_shape=jax.ShapeDtypeStruct((2, NP, HALF), jnp.float32),
    )(x, W, b.reshape(1, D))


NBUF = 6
NHALF = 6
NCHUNKH = NCHUNK // NHALF
NGROUPH = NCHUNKH // NBUF


def _sc_scatter_body(xt_hbm, src_hbm, dst_hbm, out_hbm,
                     src_v, dst_v, bufs, acc, gsem, ssem):
    c = lax.axis_index("c")
    s = lax.axis_index("s")
    seed_base = c * NP + s * RPT
    pltpu.sync_copy(xt_hbm.at[pl.ds(seed_base, RPT)],
                    acc.at[pl.ds(s * RPT, RPT)])
    plsc.subcore_barrier()

    def gather(j, b):
        return pltpu.async_copy(xt_hbm.at[src_v.at[j]], bufs.at[b],
                                gsem.at[b])

    for h in range(NHALF):
        pltpu.sync_copy(src_hbm.at[c, s, h], src_v)
        pltpu.sync_copy(dst_hbm.at[s, h], dst_v)

        for b in range(NBUF):
            gather(b, b)

        def group(g, carry):
            j0 = g * NBUF
            for b in range(NBUF):
                pltpu.make_async_copy(xt_hbm.at[src_v.at[j0 + b]],
                                      bufs.at[b], gsem.at[b]).wait()
                pltpu.async_copy(bufs.at[b], acc.at[dst_v.at[j0 + b]],
                                 ssem.at[b], add=True)
            for b in range(NBUF):
                pltpu.make_async_copy(bufs.at[b], acc.at[dst_v.at[j0 + b]],
                                      ssem.at[b]).wait()

                @pl.when(g + 1 < NGROUPH)
                def _():
                    gather(j0 + NBUF + b, b)

            return carry

        lax.fori_loop(0, NGROUPH, group, 0)

    plsc.subcore_barrier()
    pltpu.sync_copy(acc.at[pl.ds(s * RPT, RPT)],
                    out_hbm.at[pl.ds(seed_base, RPT)])


@functools.partial(jax.jit, static_argnames=())
def _sc_scatter(xt_flat, src3, dst3):
    mesh = plsc.VectorSubcoreMesh(core_axis_name="c", subcore_axis_name="s")
    run = pl.kernel(
        _sc_scatter_body,
        out_type=jax.ShapeDtypeStruct((NC * NP, HALF), jnp.float32),
        mesh=mesh,
        scratch_types=[
            pltpu.VMEM((NCHUNKH, CHUNK), jnp.int32),
            pltpu.VMEM((NCHUNKH, CHUNK), jnp.int32),
            pltpu.VMEM((NBUF, CHUNK, HALF), jnp.float32),
            pltpu.VMEM_SHARED((NP, HALF), jnp.float32),
            pltpu.SemaphoreType.DMA((NBUF,)),
            pltpu.SemaphoreType.DMA((NBUF,)),
        ],
    )
    return run(xt_flat, src3, dst3)


def _ln_body(agg_ref, g_ref, bt_ref, out_ref):
    xa = jnp.concatenate([agg_ref[0], agg_ref[1]], axis=1)
    mean = jnp.mean(xa, axis=1, keepdims=True)
    d = xa - mean
    var = jnp.mean(d * d, axis=1, keepdims=True)
    y = d * lax.rsqrt(var + 1e-5) * g_ref[0][None, :] + bt_ref[0][None, :]
    out_ref[...] = jnp.maximum(y, 0.0)


def _layernorm(agg2, gamma, beta):
    BR = 1000
    grid = (N // BR,)
    return pl.pallas_call(
        _ln_body,
        grid=grid,
        in_specs=[
            pl.BlockSpec((2, BR, HALF), lambda i: (0, i, 0)),
            pl.BlockSpec((1, D), lambda i: (0, 0)),
            pl.BlockSpec((1, D), lambda i: (0, 0)),
        ],
        out_specs=pl.BlockSpec((BR, D), lambda i: (i, 0)),
        out_shape=jax.ShapeDtypeStruct((N, D), jnp.float32),
    )(agg2, gamma.reshape(1, D), beta.reshape(1, D))


def kernel(x, edge_index, W, b, gamma, beta):
    src = edge_index[0].astype(jnp.int32)
    dst = edge_index[1].astype(jnp.int32)
    xt2 = _matmul(x, W, b)
    pad = EPTP - EPT
    pad_src = jnp.arange(pad, dtype=jnp.int32) * 37 % N
    srcp = jnp.concatenate(
        [src.reshape(NS, EPT), jnp.broadcast_to(pad_src, (NS, pad))], axis=1)
    pad_rows = N + jnp.arange(pad, dtype=jnp.int32) % (NP - N)
    dstp = jnp.concatenate(
        [dst.reshape(NS, EPT),
         jnp.broadcast_to(pad_rows, (NS, pad))], axis=1)
    src3 = jnp.stack([srcp, srcp + NP]).reshape(NC, NS, NHALF, NCHUNKH, CHUNK)
    dst3 = dstp.reshape(NS, NHALF, NCHUNKH, CHUNK)
    agg = _sc_scatter(xt2.reshape(NC * NP, HALF), src3, dst3)
    return _layernorm(agg.reshape(NC, NP, HALF), gamma, beta)


# speedup vs baseline: 1.8259x; 1.0472x over previous
import functools

import jax
import jax.numpy as jnp
from jax import lax
from jax.experimental import pallas as pl
from jax.experimental.pallas import tpu as pltpu
from jax.experimental.pallas import tpu_sc as plsc

N = 10000
E = 160000
D = 256
HALF = 128

NC = 2
NS = 16
EPT = E // NS
CHUNK = 40
NCHUNK = EPT // CHUNK
NP = 10112
RPT = NP // NS


def _matmul_body(x_ref, w_ref, b_ref, out_ref):
    y = jnp.dot(x_ref[...], w_ref[...], preferred_element_type=jnp.float32)
    y = y + b_ref[0][None, :]
    out_ref[0] = y[:, :HALF]
    out_ref[1] = y[:, HALF:]


def _matmul(x, W, b):
    BR = 1000
    grid = (N // BR,)
    return pl.pallas_call(
        _matmul_body,
        grid=grid,
        in_specs=[
            pl.BlockSpec((BR, D), lambda i: (i, 0)),
            pl.BlockSpec((D, D), lambda i: (0, 0)),
            pl.BlockSpec((1, D), lambda i: (0, 0)),
        ],
        out_specs=pl.BlockSpec((2, BR, HALF), lambda i: (0, i, 0)),
        out_shape=jax.ShapeDtypeStruct((2, NP, HALF), jnp.float32),
    )(x, W, b.reshape(1, D))


NBUF = 5
NHALF = 10
NCHUNKH = NCHUNK // NHALF
NGROUPH = NCHUNKH // NBUF


def _sc_scatter_body(xt_hbm, src_hbm, dst_hbm, out_hbm,
                     src_v, dst_v, bufs, acc, gsem, ssem, isem, seedsem):
    c = lax.axis_index("c")
    s = lax.axis_index("s")
    seed_base = c * NP + s * RPT
    seed_cp = pltpu.async_copy(xt_hbm.at[pl.ds(seed_base, RPT)],
                               acc.at[pl.ds(s * RPT, RPT)], seedsem)

    def stage_idx(h, sl):
        pltpu.async_copy(src_hbm.at[c, s, h], src_v.at[sl], isem.at[sl])
        pltpu.async_copy(dst_hbm.at[s, h], dst_v.at[sl], isem.at[sl])

    def wait_idx(h, sl):
        pltpu.make_async_copy(src_hbm.at[c, s, h], src_v.at[sl],
                              isem.at[sl]).wait()
        pltpu.make_async_copy(dst_hbm.at[s, h], dst_v.at[sl],
                              isem.at[sl]).wait()

    def gather(j, b, sl):
        return pltpu.async_copy(xt_hbm.at[src_v.at[sl, j]], bufs.at[b],
                                gsem.at[b])

    def prime(sl):
        for b in range(NBUF):
            gather(b, b, sl)

    stage_idx(0, 0)
    wait_idx(0, 0)
    prime(0)
    stage_idx(1, 1)
    seed_cp.wait()
    plsc.subcore_barrier()

    for h in range(NHALF):
        sl = h % 2

        def group(g, carry):
            j0 = g * NBUF
            for b in range(NBUF):
                pltpu.make_async_copy(xt_hbm.at[src_v.at[sl, j0 + b]],
                                      bufs.at[b], gsem.at[b]).wait()
                pltpu.async_copy(bufs.at[b], acc.at[dst_v.at[sl, j0 + b]],
                                 ssem.at[b], add=True)
            for b in range(NBUF):
                pltpu.make_async_copy(bufs.at[b],
                                      acc.at[dst_v.at[sl, j0 + b]],
                                      ssem.at[b]).wait()

                @pl.when(g + 1 < NGROUPH)
                def _():
                    gather(j0 + NBUF + b, b, sl)

            return carry

        lax.fori_loop(0, NGROUPH, group, 0)

        if h + 1 < NHALF:
            wait_idx(h + 1, 1 - sl)
            prime(1 - sl)
            if h + 2 < NHALF:
                stage_idx(h + 2, sl)

    plsc.subcore_barrier()
    pltpu.sync_copy(acc.at[pl.ds(s * RPT, RPT)],
                    out_hbm.at[pl.ds(seed_base, RPT)])


@functools.partial(jax.jit, static_argnames=())
def _sc_scatter(xt_flat, src3, dst3):
    mesh = plsc.VectorSubcoreMesh(core_axis_name="c", subcore_axis_name="s")
    run = pl.kernel(
        _sc_scatter_body,
        out_type=jax.ShapeDtypeStruct((NC * NP, HALF), jnp.float32),
        mesh=mesh,
        scratch_types=[
            pltpu.VMEM((2, NCHUNKH, CHUNK), jnp.int32),
            pltpu.VMEM((2, NCHUNKH, CHUNK), jnp.int32),
            pltpu.VMEM((NBUF, CHUNK, HALF), jnp.float32),
            pltpu.VMEM_SHARED((NP, HALF), jnp.float32),
            pltpu.SemaphoreType.DMA((NBUF,)),
            pltpu.SemaphoreType.DMA((NBUF,)),
            pltpu.SemaphoreType.DMA((2,)),
            pltpu.SemaphoreType.DMA,
        ],
    )
    return run(xt_flat, src3, dst3)


def _ln_body(agg_ref, g_ref, bt_ref, out_ref):
    xa = jnp.concatenate([agg_ref[0], agg_ref[1]], axis=1)
    mean = jnp.mean(xa, axis=1, keepdims=True)
    d = xa - mean
    var = jnp.mean(d * d, axis=1, keepdims=True)
    y = d * lax.rsqrt(var + 1e-5) * g_ref[0][None, :] + bt_ref[0][None, :]
    out_ref[...] = jnp.maximum(y, 0.0)


def _layernorm(agg2, gamma, beta):
    BR = 1000
    grid = (N // BR,)
    return pl.pallas_call(
        _ln_body,
        grid=grid,
        in_specs=[
            pl.BlockSpec((2, BR, HALF), lambda i: (0, i, 0)),
            pl.BlockSpec((1, D), lambda i: (0, 0)),
            pl.BlockSpec((1, D), lambda i: (0, 0)),
        ],
        out_specs=pl.BlockSpec((BR, D), lambda i: (i, 0)),
        out_shape=jax.ShapeDtypeStruct((N, D), jnp.float32),
    )(agg2, gamma.reshape(1, D), beta.reshape(1, D))


def kernel(x, edge_index, W, b, gamma, beta):
    src = edge_index[0].astype(jnp.int32)
    dst = edge_index[1].astype(jnp.int32)
    xt2 = _matmul(x, W, b)
    src3 = jnp.stack([src, src + NP]).reshape(NC, NS, NHALF, NCHUNKH, CHUNK)
    dst3 = dst.reshape(NS, NHALF, NCHUNKH, CHUNK)
    agg = _sc_scatter(xt2.reshape(NC * NP, HALF), src3, dst3)
    return _layernorm(agg.reshape(NC, NP, HALF), gamma, beta)


# speedup vs baseline: 1.8259x; 1.0000x over previous
import functools

import jax
import jax.numpy as jnp
from jax import lax
from jax.experimental import pallas as pl
from jax.experimental.pallas import tpu as pltpu
from jax.experimental.pallas import tpu_sc as plsc

N = 10000
E = 160000
D = 256
HALF = 128

NC = 2
NS = 16
EPT = E // NS
CHUNK = 40
NCHUNK = EPT // CHUNK
NP = 10112
RPT = NP // NS


def _matmul_body(x_ref, w_ref, b_ref, out_ref):
    y = jnp.dot(x_ref[...].astype(jnp.bfloat16), w_ref[...].astype(jnp.bfloat16),
                preferred_element_type=jnp.float32)
    y = y + b_ref[0][None, :]
    out_ref[0] = y[:, :HALF]
    out_ref[1] = y[:, HALF:]


def _matmul(x, W, b):
    BR = 1000
    grid = (N // BR,)
    return pl.pallas_call(
        _matmul_body,
        grid=grid,
        in_specs=[
            pl.BlockSpec((BR, D), lambda i: (i, 0)),
            pl.BlockSpec((D, D), lambda i: (0, 0)),
            pl.BlockSpec((1, D), lambda i: (0, 0)),
        ],
        out_specs=pl.BlockSpec((2, BR, HALF), lambda i: (0, i, 0)),
        out_shape=jax.ShapeDtypeStruct((2, NP, HALF), jnp.float32),
    )(x, W, b.reshape(1, D))


NBUF = 5
NHALF = 10
NCHUNKH = NCHUNK // NHALF
NGROUPH = NCHUNKH // NBUF


def _sc_scatter_body(xt_hbm, src_hbm, dst_hbm, out_hbm,
                     src_v, dst_v, bufs, acc, gsem, ssem, isem, seedsem):
    c = lax.axis_index("c")
    s = lax.axis_index("s")
    seed_base = c * NP + s * RPT
    seed_cp = pltpu.async_copy(xt_hbm.at[pl.ds(seed_base, RPT)],
                               acc.at[pl.ds(s * RPT, RPT)], seedsem)

    def stage_idx(h, sl):
        pltpu.async_copy(src_hbm.at[c, s, h], src_v.at[sl], isem.at[sl])
        pltpu.async_copy(dst_hbm.at[s, h], dst_v.at[sl], isem.at[sl])

    def wait_idx(h, sl):
        pltpu.make_async_copy(src_hbm.at[c, s, h], src_v.at[sl],
                              isem.at[sl]).wait()
        pltpu.make_async_copy(dst_hbm.at[s, h], dst_v.at[sl],
                              isem.at[sl]).wait()

    def gather(j, b, sl):
        return pltpu.async_copy(xt_hbm.at[src_v.at[sl, j]], bufs.at[b],
                                gsem.at[b])

    def prime(sl):
        for b in range(NBUF):
            gather(b, b, sl)

    stage_idx(0, 0)
    wait_idx(0, 0)
    prime(0)
    stage_idx(1, 1)
    seed_cp.wait()
    plsc.subcore_barrier()

    for h in range(NHALF):
        sl = h % 2

        def group(g, carry):
            j0 = g * NBUF
            for b in range(NBUF):
                pltpu.make_async_copy(xt_hbm.at[src_v.at[sl, j0 + b]],
                                      bufs.at[b], gsem.at[b]).wait()
                pltpu.async_copy(bufs.at[b], acc.at[dst_v.at[sl, j0 + b]],
                                 ssem.at[b], add=True)
            for b in range(NBUF):
                pltpu.make_async_copy(bufs.at[b],
                                      acc.at[dst_v.at[sl, j0 + b]],
                                      ssem.at[b]).wait()

                @pl.when(g + 1 < NGROUPH)
                def _():
                    gather(j0 + NBUF + b, b, sl)

            return carry

        lax.fori_loop(0, NGROUPH, group, 0)

        if h + 1 < NHALF:
            wait_idx(h + 1, 1 - sl)
            prime(1 - sl)
            if h + 2 < NHALF:
                stage_idx(h + 2, sl)

    plsc.subcore_barrier()
    pltpu.sync_copy(acc.at[pl.ds(s * RPT, RPT)],
                    out_hbm.at[pl.ds(seed_base, RPT)])


@functools.partial(jax.jit, static_argnames=())
def _sc_scatter(xt_flat, src3, dst3):
    mesh = plsc.VectorSubcoreMesh(core_axis_name="c", subcore_axis_name="s")
    run = pl.kernel(
        _sc_scatter_body,
        out_type=jax.ShapeDtypeStruct((NC * NP, HALF), jnp.float32),
        mesh=mesh,
        scratch_types=[
            pltpu.VMEM((2, NCHUNKH, CHUNK), jnp.int32),
            pltpu.VMEM((2, NCHUNKH, CHUNK), jnp.int32),
            pltpu.VMEM((NBUF, CHUNK, HALF), jnp.float32),
            pltpu.VMEM_SHARED((NP, HALF), jnp.float32),
            pltpu.SemaphoreType.DMA((NBUF,)),
            pltpu.SemaphoreType.DMA((NBUF,)),
            pltpu.SemaphoreType.DMA((2,)),
            pltpu.SemaphoreType.DMA,
        ],
    )
    return run(xt_flat, src3, dst3)


def _ln_body(agg_ref, g_ref, bt_ref, out_ref):
    xa = jnp.concatenate([agg_ref[0], agg_ref[1]], axis=1)
    mean = jnp.mean(xa, axis=1, keepdims=True)
    d = xa - mean
    var = jnp.mean(d * d, axis=1, keepdims=True)
    y = d * lax.rsqrt(var + 1e-5) * g_ref[0][None, :] + bt_ref[0][None, :]
    out_ref[...] = jnp.maximum(y, 0.0)


def _layernorm(agg2, gamma, beta):
    BR = 1000
    grid = (N // BR,)
    return pl.pallas_call(
        _ln_body,
        grid=grid,
        in_specs=[
            pl.BlockSpec((2, BR, HALF), lambda i: (0, i, 0)),
            pl.BlockSpec((1, D), lambda i: (0, 0)),
            pl.BlockSpec((1, D), lambda i: (0, 0)),
        ],
        out_specs=pl.BlockSpec((BR, D), lambda i: (i, 0)),
        out_shape=jax.ShapeDtypeStruct((N, D), jnp.float32),
    )(agg2, gamma.reshape(1, D), beta.reshape(1, D))


def kernel(x, edge_index, W, b, gamma, beta):
    src = edge_index[0].astype(jnp.int32)
    dst = edge_index[1].astype(jnp.int32)
    xt2 = _matmul(x, W, b)
    src3 = jnp.stack([src, src + NP]).reshape(NC, NS, NHALF, NCHUNKH, CHUNK)
    dst3 = dst.reshape(NS, NHALF, NCHUNKH, CHUNK)
    agg = _sc_scatter(xt2.reshape(NC * NP, HALF), src3, dst3)
    return _layernorm(agg.reshape(NC, NP, HALF), gamma, beta)


# speedup vs baseline: 1.8737x; 1.0262x over previous
import functools

import jax
import jax.numpy as jnp
from jax import lax
from jax.experimental import pallas as pl
from jax.experimental.pallas import tpu as pltpu
from jax.experimental.pallas import tpu_sc as plsc

N = 10000
E = 160000
D = 256
HALF = 128

NC = 2
NS = 16
EPT = E // NS
CHUNK = 40
NCHUNK = EPT // CHUNK
NP = 10112
RPT = NP // NS


def _matmul_body(x_ref, w_ref, b_ref, out_ref):
    y = jnp.dot(x_ref[...], w_ref[...], preferred_element_type=jnp.float32)
    y = y + b_ref[0][None, :]
    out_ref[0] = y[:, :HALF]
    out_ref[1] = y[:, HALF:]


def _matmul(x, W, b):
    BR = 1000
    grid = (N // BR,)
    return pl.pallas_call(
        _matmul_body,
        grid=grid,
        in_specs=[
            pl.BlockSpec((BR, D), lambda i: (i, 0)),
            pl.BlockSpec((D, D), lambda i: (0, 0)),
            pl.BlockSpec((1, D), lambda i: (0, 0)),
        ],
        out_specs=pl.BlockSpec((2, BR, HALF), lambda i: (0, i, 0)),
        out_shape=jax.ShapeDtypeStruct((2, NP, HALF), jnp.float32),
    )(x, W, b.reshape(1, D))


NBUF = 5
NHALF = 10
NCHUNKH = NCHUNK // NHALF
NGROUPH = NCHUNKH // NBUF


def _sc_scatter_body(xt_hbm, src_hbm, dst_hbm, out_hbm,
                     src_v, dst_v, bufs, acc, gsem, ssem, isem, seedsem):
    c = lax.axis_index("c")
    s = lax.axis_index("s")
    seed_base = c * NP + s * RPT
    seed_cp = pltpu.async_copy(xt_hbm.at[pl.ds(seed_base, RPT)],
                               acc.at[pl.ds(s * RPT, RPT)], seedsem)

    def stage_idx(h, sl):
        pltpu.async_copy(src_hbm.at[c, s, h], src_v.at[sl], isem.at[sl])
        pltpu.async_copy(dst_hbm.at[s, h], dst_v.at[sl], isem.at[sl])

    def wait_idx(h, sl):
        pltpu.make_async_copy(src_hbm.at[c, s, h], src_v.at[sl],
                              isem.at[sl]).wait()
        pltpu.make_async_copy(dst_hbm.at[s, h], dst_v.at[sl],
                              isem.at[sl]).wait()

    def gather(j, b, sl):
        return pltpu.async_copy(xt_hbm.at[src_v.at[sl, j]], bufs.at[b],
                                gsem.at[b])

    def prime(sl):
        for b in range(NBUF):
            gather(b, b, sl)

    stage_idx(0, 0)
    wait_idx(0, 0)
    prime(0)
    stage_idx(1, 1)
    seed_cp.wait()
    plsc.subcore_barrier()

    for h in range(NHALF):
        sl = h % 2

        def group(g, carry):
            j0 = g * NBUF
            for b in range(NBUF):
                pltpu.make_async_copy(xt_hbm.at[src_v.at[sl, j0 + b]],
                                      bufs.at[b], gsem.at[b]).wait()
                pltpu.async_copy(bufs.at[b], acc.at[dst_v.at[sl, j0 + b]],
                                 ssem.at[b], add=True)
            for b in range(NBUF):
                pltpu.make_async_copy(bufs.at[b],
                                      acc.at[dst_v.at[sl, j0 + b]],
                                      ssem.at[b]).wait()
                gather(j0 + NBUF + b, b, sl)
            return carry

        lax.fori_loop(0, NGROUPH - 1, group, 0)

        jlast = (NGROUPH - 1) * NBUF
        for b in range(NBUF):
            pltpu.make_async_copy(xt_hbm.at[src_v.at[sl, jlast + b]],
                                  bufs.at[b], gsem.at[b]).wait()
            pltpu.async_copy(bufs.at[b], acc.at[dst_v.at[sl, jlast + b]],
                             ssem.at[b], add=True)
        if h + 1 < NHALF:
            wait_idx(h + 1, 1 - sl)
        for b in range(NBUF):
            pltpu.make_async_copy(bufs.at[b], acc.at[dst_v.at[sl, jlast + b]],
                                  ssem.at[b]).wait()
            if h + 1 < NHALF:
                gather(b, b, 1 - sl)
        if h + 2 < NHALF:
            stage_idx(h + 2, sl)

    plsc.subcore_barrier()
    pltpu.sync_copy(acc.at[pl.ds(s * RPT, RPT)],
                    out_hbm.at[pl.ds(seed_base, RPT)])


@functools.partial(jax.jit, static_argnames=())
def _sc_scatter(xt_flat, src3, dst3):
    mesh = plsc.VectorSubcoreMesh(core_axis_name="c", subcore_axis_name="s")
    run = pl.kernel(
        _sc_scatter_body,
        out_type=jax.ShapeDtypeStruct((NC * NP, HALF), jnp.float32),
        mesh=mesh,
        scratch_types=[
            pltpu.VMEM((2, NCHUNKH, CHUNK), jnp.int32),
            pltpu.VMEM((2, NCHUNKH, CHUNK), jnp.int32),
            pltpu.VMEM((NBUF, CHUNK, HALF), jnp.float32),
            pltpu.VMEM_SHARED((NP, HALF), jnp.float32),
            pltpu.SemaphoreType.DMA((NBUF,)),
            pltpu.SemaphoreType.DMA((NBUF,)),
            pltpu.SemaphoreType.DMA((2,)),
            pltpu.SemaphoreType.DMA,
        ],
    )
    return run(xt_flat, src3, dst3)


def _ln_body(agg_ref, g_ref, bt_ref, out_ref):
    xa = jnp.concatenate([agg_ref[0], agg_ref[1]], axis=1)
    mean = jnp.mean(xa, axis=1, keepdims=True)
    d = xa - mean
    var = jnp.mean(d * d, axis=1, keepdims=True)
    y = d * lax.rsqrt(var + 1e-5) * g_ref[0][None, :] + bt_ref[0][None, :]
    out_ref[...] = jnp.maximum(y, 0.0)


def _layernorm(agg2, gamma, beta):
    BR = 1000
    grid = (N // BR,)
    return pl.pallas_call(
        _ln_body,
        grid=grid,
        in_specs=[
            pl.BlockSpec((2, BR, HALF), lambda i: (0, i, 0)),
            pl.BlockSpec((1, D), lambda i: (0, 0)),
            pl.BlockSpec((1, D), lambda i: (0, 0)),
        ],
        out_specs=pl.BlockSpec((BR, D), lambda i: (i, 0)),
        out_shape=jax.ShapeDtypeStruct((N, D), jnp.float32),
    )(agg2, gamma.reshape(1, D), beta.reshape(1, D))


def kernel(x, edge_index, W, b, gamma, beta):
    src = edge_index[0].astype(jnp.int32)
    dst = edge_index[1].astype(jnp.int32)
    xt2 = _matmul(x, W, b)
    src3 = jnp.stack([src, src + NP]).reshape(NC, NS, NHALF, NCHUNKH, CHUNK)
    dst3 = dst.reshape(NS, NHALF, NCHUNKH, CHUNK)
    agg = _sc_scatter(xt2.reshape(NC * NP, HALF), src3, dst3)
    return _layernorm(agg.reshape(NC, NP, HALF), gamma, beta)
